# Initial kernel scaffold; baseline (speedup 1.0000x reference)
import jax
import jax.numpy as jnp
from jax.experimental import pallas as pl


def kernel(x, beta_u, beta_i, alfa):
    raise NotImplementedError("write your pallas kernel here")


# speedup vs baseline: 1.0484x; 1.0484x over previous
import jax
import jax.numpy as jnp
from jax import lax
from jax.experimental import pallas as pl
from jax.experimental.pallas import tpu as pltpu
from jax.experimental.pallas import tpu_sc as plsc

BATCH = 16384
LANES = 16
CHUNK = 128

_info = plsc.get_sparse_core_info()
_NC, _NS = _info.num_cores, _info.num_subcores
NW = _NC * _NS
BPW = BATCH // NW
NCH = BPW // CHUNK


def _sc_body(xu_hbm, xi_hbm, bu_hbm, bi_hbm, alfa_hbm, out_hbm,
             idxu_v, idxi_v, u_v, i_v, a_v, sem):
    wid = lax.axis_index("s") * _NC + lax.axis_index("c")
    base = wid * BPW

    pltpu.sync_copy(xu_hbm.at[pl.ds(base, BPW)], idxu_v)
    pltpu.sync_copy(xi_hbm.at[pl.ds(base, BPW)], idxi_v)
    pltpu.sync_copy(alfa_hbm, a_v)

    copies = []
    for j in range(NCH):
        sl = pl.ds(j * CHUNK, CHUNK)
        copies.append(pltpu.async_copy(bu_hbm.at[idxu_v.at[sl]], u_v.at[sl], sem))
        copies.append(pltpu.async_copy(bi_hbm.at[idxi_v.at[sl]], i_v.at[sl], sem))
    for c in copies:
        c.wait()

    av = a_v[...]
    for j in range(BPW // LANES):
        sl = pl.ds(j * LANES, LANES)
        u_v[sl] = u_v[sl] + i_v[sl] + av

    pltpu.sync_copy(u_v, out_hbm.at[pl.ds(base, BPW)])


@jax.jit
def _run(xu, xi, bu, bi, alfa_b):
    mesh = plsc.VectorSubcoreMesh(core_axis_name="c", subcore_axis_name="s")
    f = pl.kernel(
        _sc_body,
        out_type=jax.ShapeDtypeStruct((BATCH,), jnp.float32),
        mesh=mesh,
        scratch_types=[
            pltpu.VMEM((BPW,), jnp.int32),
            pltpu.VMEM((BPW,), jnp.int32),
            pltpu.VMEM((BPW,), jnp.float32),
            pltpu.VMEM((BPW,), jnp.float32),
            pltpu.VMEM((LANES,), jnp.float32),
            pltpu.SemaphoreType.DMA,
        ],
    )
    return f(xu, xi, bu, bi, alfa_b)


def kernel(x, beta_u, beta_i, alfa):
    x = x.astype(jnp.int32)
    xu = x[:, 0]
    xi = x[:, 1]
    bu = beta_u.reshape((beta_u.shape[0],))
    bi = beta_i.reshape((beta_i.shape[0],))
    alfa_b = jnp.broadcast_to(alfa.reshape(()), (LANES,))
    out = _run(xu, xi, bu, bi, alfa_b)
    return out.reshape((BATCH, 1))


# speedup vs baseline: 1.0491x; 1.0006x over previous
import jax
import jax.numpy as jnp
from jax import lax
from jax.experimental import pallas as pl
from jax.experimental.pallas import tpu as pltpu
from jax.experimental.pallas import tpu_sc as plsc

BATCH = 16384
LANES = 16
CHUNK = 128

_info = plsc.get_sparse_core_info()
_NC, _NS = _info.num_cores, _info.num_subcores
NW = _NC * _NS
BPW = BATCH // NW
NCH = BPW // CHUNK


def _gather_body(idx_hbm, tab_hbm, base_hbm, out_hbm,
                 idx_v, g_v, b_v, sem):
    wid = lax.axis_index("s") * _NC + lax.axis_index("c")
    base = wid * BPW
    splat = base_hbm.shape[0] == LANES

    pltpu.sync_copy(idx_hbm.at[pl.ds(base, BPW)], idx_v)
    if splat:
        pltpu.sync_copy(base_hbm, b_v)
    else:
        pltpu.sync_copy(base_hbm.at[pl.ds(base, BPW)], b_v)

    copies = []
    for j in range(NCH):
        sl = pl.ds(j * CHUNK, CHUNK)
        copies.append(pltpu.async_copy(tab_hbm.at[idx_v.at[sl]], g_v.at[sl], sem))
    for c in copies:
        c.wait()

    if splat:
        bv = b_v[...]
        for j in range(BPW // LANES):
            sl = pl.ds(j * LANES, LANES)
            g_v[sl] = g_v[sl] + bv
    else:
        for j in range(BPW // LANES):
            sl = pl.ds(j * LANES, LANES)
            g_v[sl] = g_v[sl] + b_v[sl]

    pltpu.sync_copy(g_v, out_hbm.at[pl.ds(base, BPW)])


def _make_gather(base_len):
    mesh = plsc.VectorSubcoreMesh(core_axis_name="c", subcore_axis_name="s")
    return pl.kernel(
        _gather_body,
        out_type=jax.ShapeDtypeStruct((BATCH,), jnp.float32),
        mesh=mesh,
        scratch_types=[
            pltpu.VMEM((BPW,), jnp.int32),
            pltpu.VMEM((BPW,), jnp.float32),
            pltpu.VMEM((base_len,), jnp.float32),
            pltpu.SemaphoreType.DMA,
        ],
    )


@jax.jit
def _run(x, bu, bi, alfa):
    bu_flat = bu.reshape((bu.shape[0],))
    bi_flat = bi.reshape((bi.shape[0],))
    alfa_b = jnp.broadcast_to(alfa.reshape(()), (LANES,))
    partial = _make_gather(LANES)(x[:, 0], bu_flat, alfa_b)
    out = _make_gather(BPW)(x[:, 1], bi_flat, partial)
    return out.reshape((BATCH, 1))


def kernel(x, beta_u, beta_i, alfa):
    return _run(x.astype(jnp.int32), beta_u, beta_i, alfa)


# speedup vs baseline: 1.0523x; 1.0030x over previous
import jax
import jax.numpy as jnp
from jax import lax
from jax.experimental import pallas as pl
from jax.experimental.pallas import tpu as pltpu
from jax.experimental.pallas import tpu_sc as plsc

BATCH = 16384
LANES = 16
CHUNK = 128

_info = plsc.get_sparse_core_info()
_NC, _NS = _info.num_cores, _info.num_subcores
NW = _NC * _NS
BPW = BATCH // NW
NCH = BPW // CHUNK


def _gather_body(idx_hbm, tab_hbm, base_hbm, out_hbm,
                 idx_v, g_v, b_v, sem, bsem):
    wid = lax.axis_index("s") * _NC + lax.axis_index("c")
    base = wid * BPW
    splat = base_hbm.shape[0] == LANES

    if splat:
        bcopy = pltpu.make_async_copy(base_hbm, b_v, bsem)
    else:
        bcopy = pltpu.make_async_copy(base_hbm.at[pl.ds(base, BPW)], b_v, bsem)
    bcopy.start()
    pltpu.sync_copy(idx_hbm.at[pl.ds(base, BPW)], idx_v)

    copies = []
    for j in range(NCH):
        sl = pl.ds(j * CHUNK, CHUNK)
        copies.append(pltpu.async_copy(tab_hbm.at[idx_v.at[sl]], g_v.at[sl], sem))
    for c in copies:
        c.wait()
    bcopy.wait()

    if splat:
        bv = b_v[...]
        for j in range(BPW // LANES):
            sl = pl.ds(j * LANES, LANES)
            g_v[sl] = g_v[sl] + bv
    else:
        for j in range(BPW // LANES):
            sl = pl.ds(j * LANES, LANES)
            g_v[sl] = g_v[sl] + b_v[sl]

    pltpu.sync_copy(g_v, out_hbm.at[pl.ds(base, BPW)])


def _make_gather(base_len):
    mesh = plsc.VectorSubcoreMesh(core_axis_name="c", subcore_axis_name="s")
    return pl.kernel(
        _gather_body,
        out_type=jax.ShapeDtypeStruct((BATCH,), jnp.float32),
        mesh=mesh,
        scratch_types=[
            pltpu.VMEM((BPW,), jnp.int32),
            pltpu.VMEM((BPW,), jnp.float32),
            pltpu.VMEM((base_len,), jnp.float32),
            pltpu.SemaphoreType.DMA,
            pltpu.SemaphoreType.DMA,
        ],
    )


@jax.jit
def _run(x, bu, bi, alfa):
    bu_flat = bu.reshape((bu.shape[0],))
    bi_flat = bi.reshape((bi.shape[0],))
    alfa_b = jnp.broadcast_to(alfa.reshape(()), (LANES,))
    partial = _make_gather(LANES)(x[:, 0], bu_flat, alfa_b)
    out = _make_gather(BPW)(x[:, 1], bi_flat, partial)
    return out.reshape((BATCH, 1))


def kernel(x, beta_u, beta_i, alfa):
    return _run(x.astype(jnp.int32), beta_u, beta_i, alfa)
